# baseline (speedup 1.0000x reference)
import functools

import jax
import jax.numpy as jnp
from jax import lax
from jax.experimental import pallas as pl
from jax.experimental.pallas import tpu as pltpu
from jax.experimental.pallas import tpu_sc as plsc

_SIZE = 100000
_N = 1024
_SMOOTHING = 0.1
_CONF = 1.0 - _SMOOTHING
_S = _SMOOTHING / (_SIZE - 2)
import math as _math

_C1 = _CONF * _math.log(_CONF) + _SMOOTHING * _math.log(_S)


_BC = 2048
_NB = -(-_SIZE // _BC)


def _tc_body(t_ref, x_ref, o_ref):
    k = pl.program_id(0)
    xb = x_ref[...]
    m = (t_ref[...] != 0).astype(jnp.float32)

    def wsum(xm):
        rows = jnp.sum(xm, axis=1, keepdims=True)
        return jnp.sum(rows * m)

    edge = jnp.logical_or(k == 0, k == _NB - 1)

    @pl.when(edge)
    def _():
        col = lax.broadcasted_iota(jnp.int32, xb.shape, 1) + k * _BC
        valid = jnp.logical_and(col >= 1, col < _SIZE)
        part = -_S * wsum(jnp.where(valid, xb, 0.0))
        part = part + jnp.where(k == 0, _C1 * jnp.sum(m), 0.0)

        part2d = jnp.reshape(part, (1, 1))

        @pl.when(k == 0)
        def _():
            o_ref[...] = part2d

        @pl.when(k != 0)
        def _():
            o_ref[...] += part2d

    @pl.when(jnp.logical_not(edge))
    def _():
        o_ref[...] += jnp.reshape(-_S * wsum(xb), (1, 1))


def _tc_reduce(x, t2d):
    return pl.pallas_call(
        _tc_body,
        grid=(_NB,),
        in_specs=[
            pl.BlockSpec((_N, 1), lambda k: (0, 0)),
            pl.BlockSpec((_N, _BC), lambda k: (0, k)),
        ],
        out_specs=pl.BlockSpec((1, 1), lambda k: (0, 0)),
        out_shape=jax.ShapeDtypeStruct((1, 1), jnp.float32),
        compiler_params=pltpu.CompilerParams(
            dimension_semantics=("arbitrary",),
        ),
    )(t2d, x)


_L = 16


def _make_sc_gather(nw):
    bpw = _N // nw
    mesh = plsc.VectorSubcoreMesh(core_axis_name="c", subcore_axis_name="s")
    info = plsc.get_sparse_core_info()
    nc = info.num_cores

    @functools.partial(
        pl.kernel,
        mesh=mesh,
        out_type=jax.ShapeDtypeStruct((nw, _L), jnp.float32),
        scratch_types=[
            pltpu.VMEM((bpw,), jnp.int32),
            pltpu.VMEM((bpw,), jnp.int32),
            pltpu.VMEM((bpw,), jnp.float32),
            pltpu.VMEM((_L,), jnp.float32),
            pltpu.SemaphoreType.DMA,
        ],
    )
    def sc_gather(tgt_hbm, xflat_hbm, out_hbm, tgt_v, idx_v, val_v, acc_v, sem):
        wid = lax.axis_index("s") * nc + lax.axis_index("c")
        base = wid * bpw
        pltpu.sync_copy(tgt_hbm.at[pl.ds(base, bpw)], tgt_v)
        for j in range(bpw // _L):
            sl = pl.ds(j * _L, _L)
            row = lax.iota(jnp.int32, 16) + (base + j * _L)
            idx_v[sl] = row * _SIZE + tgt_v[sl]
        pltpu.async_copy(xflat_hbm.at[idx_v], val_v, sem).wait()
        acc = jnp.zeros((_L,), jnp.float32)
        for j in range(bpw // _L):
            sl = pl.ds(j * _L, _L)
            acc = acc + jnp.where(tgt_v[sl] == 0, 0.0, val_v[sl])
        acc_v[...] = acc
        pltpu.sync_copy(acc_v, out_hbm.at[wid])

    return sc_gather


def kernel(x, target):
    t32 = target.astype(jnp.int32)
    t2d = t32.reshape(_N, 1)
    tc_parts = _tc_reduce(x, t2d)

    info = plsc.get_sparse_core_info()
    nw = info.num_cores * info.num_subcores
    xflat = x.reshape(_N * _SIZE)
    sc_parts = _make_sc_gather(nw)(t32, xflat)

    return jnp.sum(tc_parts) - jnp.float32(_CONF - _S) * jnp.sum(sc_parts)


# speedup vs baseline: 2.1562x; 2.1562x over previous
import functools

import jax
import jax.numpy as jnp
from jax import lax
from jax.experimental import pallas as pl
from jax.experimental.pallas import tpu as pltpu
from jax.experimental.pallas import tpu_sc as plsc

_SIZE = 100000
_N = 1024
_SMOOTHING = 0.1
_CONF = 1.0 - _SMOOTHING
_S = _SMOOTHING / (_SIZE - 2)
import math as _math

_C1 = _CONF * _math.log(_CONF) + _SMOOTHING * _math.log(_S)


_BC = 2048
_NB = -(-_SIZE // _BC)


def _tc_body(t_ref, x_ref, o_ref):
    k = pl.program_id(0)
    xb = x_ref[...]
    m = (t_ref[...] != 0).astype(jnp.float32)

    def wsum(xm):
        rows = jnp.sum(xm, axis=1, keepdims=True)
        return jnp.sum(rows * m)

    edge = jnp.logical_or(k == 0, k == _NB - 1)

    @pl.when(edge)
    def _():
        col = lax.broadcasted_iota(jnp.int32, xb.shape, 1) + k * _BC
        valid = jnp.logical_and(col >= 1, col < _SIZE)
        part = -_S * wsum(jnp.where(valid, xb, 0.0))
        part = part + jnp.where(k == 0, _C1 * jnp.sum(m), 0.0)

        part2d = jnp.reshape(part, (1, 1))

        @pl.when(k == 0)
        def _():
            o_ref[...] = part2d

        @pl.when(k != 0)
        def _():
            o_ref[...] += part2d

    @pl.when(jnp.logical_not(edge))
    def _():
        o_ref[...] += jnp.reshape(-_S * wsum(xb), (1, 1))


def _tc_reduce(x, t2d):
    return pl.pallas_call(
        _tc_body,
        grid=(_NB,),
        in_specs=[
            pl.BlockSpec((_N, 1), lambda k: (0, 0)),
            pl.BlockSpec((_N, _BC), lambda k: (0, k)),
        ],
        out_specs=pl.BlockSpec((1, 1), lambda k: (0, 0)),
        out_shape=jax.ShapeDtypeStruct((1, 1), jnp.float32),
        compiler_params=pltpu.CompilerParams(
            dimension_semantics=("arbitrary",),
        ),
    )(t2d, x)


_L = 16


def _make_sc_gather(nw):
    bpw = _N // nw
    mesh = plsc.VectorSubcoreMesh(core_axis_name="c", subcore_axis_name="s")
    info = plsc.get_sparse_core_info()
    nc = info.num_cores

    @functools.partial(
        pl.kernel,
        mesh=mesh,
        out_type=jax.ShapeDtypeStruct((nw, _L), jnp.float32),
        scratch_types=[
            pltpu.VMEM((bpw,), jnp.int32),
            pltpu.VMEM((bpw,), jnp.int32),
            pltpu.VMEM((bpw,), jnp.float32),
            pltpu.VMEM((_L,), jnp.float32),
            pltpu.SemaphoreType.DMA,
        ],
    )
    def sc_gather(tgt_hbm, xflat_hbm, out_hbm, tgt_v, idx_v, val_v, acc_v, sem):
        wid = lax.axis_index("s") * nc + lax.axis_index("c")
        base = wid * bpw
        pltpu.sync_copy(tgt_hbm.at[pl.ds(base, bpw)], tgt_v)
        for j in range(bpw // _L):
            sl = pl.ds(j * _L, _L)
            row = lax.iota(jnp.int32, 16) + (base + j * _L)
            idx_v[sl] = row * _SIZE + tgt_v[sl]
        pltpu.async_copy(xflat_hbm.at[idx_v], val_v, sem).wait()
        acc = jnp.zeros((_L,), jnp.float32)
        for j in range(bpw // _L):
            sl = pl.ds(j * _L, _L)
            acc = acc + jnp.where(tgt_v[sl] == 0, 0.0, val_v[sl])
        acc_v[...] = acc
        pltpu.sync_copy(acc_v, out_hbm.at[wid])

    return sc_gather


def kernel(x, target):
    t32 = target.astype(jnp.int32)
    t2d = t32.reshape(_N, 1)
    tc_parts = _tc_reduce(x, t2d)

    g = jnp.sum(jnp.where(t32 != 0, x[jnp.arange(_N), t32], 0.0))

    return jnp.sum(tc_parts) - jnp.float32(_CONF - _S) * g
